# baseline (speedup 1.0000x reference)
import functools

import jax
import jax.numpy as jnp
from jax import lax
from jax.experimental import pallas as pl
from jax.experimental.pallas import tpu as pltpu
from jax.experimental.pallas import tpu_sc as plsc

B = 4096
T = 125
D = 64
NC = 2
NS = 16
NW = NC * NS
BPW = B // NW
KV = D // 16


def _body(smis, char, zeo2, syn2, pe2, te,
          out, zeo_o, syn_o,
          idx_v, buf_v, obuf_v, pe_v, te_v, zs_v, gsem, osem):
    cid = lax.axis_index("c")
    sid = lax.axis_index("s")
    wid = sid * NC + cid
    base = wid * BPW

    pltpu.sync_copy(smis.at[pl.ds(base, BPW)], idx_v)
    pltpu.sync_copy(pe2, pe_v)
    pltpu.sync_copy(te, te_v)

    def pe_row(pr, c):
        for k in range(KV):
            sl = pl.ds(k * 16, 16)
            pe_v[pr, sl] = pe_v[pr, sl] + te_v[2, sl]
        return c
    lax.fori_loop(0, T, pe_row, 0)

    for src, dst, trow in ((zeo2, zeo_o, 0), (syn2, syn_o, 1)):
        pltpu.sync_copy(src.at[pl.ds(base, BPW)], zs_v)

        def zrow(i, c, trow=trow):
            for k in range(KV):
                sl = pl.ds(k * 16, 16)
                zs_v[i, sl] = zs_v[i, sl] + te_v[trow, sl]
            return c
        lax.fori_loop(0, BPW, zrow, 0)
        pltpu.sync_copy(zs_v, dst.at[pl.ds(base, BPW)])

    def g_start(r, b):
        pltpu.make_async_copy(char.at[idx_v.at[r]], buf_v.at[b],
                              gsem.at[b]).start()

    def g_wait(r, b):
        pltpu.make_async_copy(char.at[idx_v.at[r]], buf_v.at[b],
                              gsem.at[b]).wait()

    def o_start(r, b):
        pltpu.make_async_copy(obuf_v.at[b], out.at[base + r],
                              osem.at[b]).start()

    def o_wait(r, b):
        pltpu.make_async_copy(obuf_v.at[b], out.at[base + r],
                              osem.at[b]).wait()

    def add_rows(b):
        def add_row(pr, cc):
            for k in range(KV):
                sl = pl.ds(k * 16, 16)
                obuf_v[b, pr, sl] = buf_v[b, pr, sl] + pe_v[pr, sl]
            return cc
        lax.fori_loop(0, T, add_row, 0, unroll=5)

    for b in range(2):
        g_start(b, b)
    for b in range(2):
        g_wait(b, b)
        add_rows(b)
        g_start(2 + b, b)
        o_start(b, b)

    def pair(j, c):
        for b in range(2):
            r = 2 * j + b
            o_wait(r - 2, b)
            g_wait(r, b)
            add_rows(b)

            @pl.when(j < BPW // 2 - 1)
            def _():
                g_start(r + 2, b)
            o_start(r, b)
        return c
    lax.fori_loop(1, BPW // 2, pair, 0)
    for b in range(2):
        o_wait(BPW - 2 + b, b)


@functools.partial(
    pl.kernel,
    mesh=plsc.VectorSubcoreMesh(core_axis_name="c", subcore_axis_name="s"),
    compiler_params=pltpu.CompilerParams(use_tc_tiling_on_sc=False),
    out_type=[
        jax.ShapeDtypeStruct((B, T, D), jnp.float32),
        jax.ShapeDtypeStruct((B, D), jnp.float32),
        jax.ShapeDtypeStruct((B, D), jnp.float32),
    ],
    scratch_types=[
        pltpu.VMEM((BPW, T), jnp.int32),
        pltpu.VMEM((2, T, D), jnp.float32),
        pltpu.VMEM((2, T, D), jnp.float32),
        pltpu.VMEM((T, D), jnp.float32),
        pltpu.VMEM((3, D), jnp.float32),
        pltpu.VMEM((BPW, D), jnp.float32),
        pltpu.SemaphoreType.DMA((2,)),
        pltpu.SemaphoreType.DMA((2,)),
    ],
)
def _embed(smis, char, zeo2, syn2, pe2, te, out, zeo_o, syn_o,
           idx_v, buf_v, obuf_v, pe_v, te_v, zs_v, gsem, osem):
    _body(smis, char, zeo2, syn2, pe2, te, out, zeo_o, syn_o,
          idx_v, buf_v, obuf_v, pe_v, te_v, zs_v, gsem, osem)


def kernel(zeo, syn, smis_seq, char_embed, type_embed, pe):
    b, t = smis_seq.shape
    d = char_embed.shape[1]
    zeo2 = zeo.reshape(b, d)
    syn2 = syn.reshape(b, d)
    pe2 = pe.reshape(t, d)
    out, zeo_o, syn_o = _embed(smis_seq, char_embed, zeo2, syn2, pe2,
                               type_embed)
    return out, zeo_o.reshape(b, 1, d), syn_o.reshape(b, 1, d)


# speedup vs baseline: 1.4359x; 1.4359x over previous
import functools

import jax
import jax.numpy as jnp
from jax import lax
from jax.experimental import pallas as pl
from jax.experimental.pallas import tpu as pltpu
from jax.experimental.pallas import tpu_sc as plsc

B = 4096
T = 125
D = 64
NC = 2
NS = 16
NW = NC * NS
BPW = B // NW
KV = D // 16


def _body(smis, char, zeo2, syn2, pe2, te,
          out, zeo_o, syn_o,
          idx_v, buf_v, obuf_v, pe_v, te_v, zs_v, gsem, osem):
    cid = lax.axis_index("c")
    sid = lax.axis_index("s")
    wid = sid * NC + cid
    base = wid * BPW

    pltpu.sync_copy(smis.at[pl.ds(base, BPW)], idx_v)
    pltpu.sync_copy(pe2, pe_v)
    pltpu.sync_copy(te, te_v)

    def pe_row(pr, c):
        for k in range(KV):
            sl = pl.ds(k * 16, 16)
            pe_v[pr, sl] = pe_v[pr, sl] + te_v[2, sl]
        return c
    lax.fori_loop(0, T, pe_row, 0)

    for src, dst, trow in ((zeo2, zeo_o, 0), (syn2, syn_o, 1)):
        pltpu.sync_copy(src.at[pl.ds(base, BPW)], zs_v)

        def zrow(i, c, trow=trow):
            for k in range(KV):
                sl = pl.ds(k * 16, 16)
                zs_v[i, sl] = zs_v[i, sl] + te_v[trow, sl]
            return c
        lax.fori_loop(0, BPW, zrow, 0)
        pltpu.sync_copy(zs_v, dst.at[pl.ds(base, BPW)])

    def g_start(r, b):
        pltpu.make_async_copy(char.at[idx_v.at[r]], buf_v.at[b],
                              gsem.at[b]).start()

    def g_wait(r, b):
        pltpu.make_async_copy(char.at[idx_v.at[r]], buf_v.at[b],
                              gsem.at[b]).wait()

    def o_start(r, b):
        pltpu.make_async_copy(obuf_v.at[b], out.at[base + r],
                              osem.at[b]).start()

    def o_wait(r, b):
        pltpu.make_async_copy(obuf_v.at[b], out.at[base + r],
                              osem.at[b]).wait()

    def add_rows(b):
        def add_row(pr, cc):
            for k in range(KV):
                sl = pl.ds(k * 16, 16)
                obuf_v[b, pr, sl] = buf_v[b, pr, sl] + pe_v[pr, sl]
            return cc
        lax.fori_loop(0, T, add_row, 0)

    for b in range(2):
        g_start(b, b)
    for b in range(2):
        g_wait(b, b)
        add_rows(b)
        g_start(2 + b, b)
        o_start(b, b)

    def pair(j, c):
        for b in range(2):
            r = 2 * j + b
            o_wait(r - 2, b)
            g_wait(r, b)
            add_rows(b)

            @pl.when(j < BPW // 2 - 1)
            def _():
                g_start(r + 2, b)
            o_start(r, b)
        return c
    lax.fori_loop(1, BPW // 2, pair, 0)
    for b in range(2):
        o_wait(BPW - 2 + b, b)


@functools.partial(
    pl.kernel,
    mesh=plsc.VectorSubcoreMesh(core_axis_name="c", subcore_axis_name="s"),
    compiler_params=pltpu.CompilerParams(use_tc_tiling_on_sc=False),
    out_type=[
        jax.ShapeDtypeStruct((B, T, D), jnp.float32),
        jax.ShapeDtypeStruct((B, D), jnp.float32),
        jax.ShapeDtypeStruct((B, D), jnp.float32),
    ],
    scratch_types=[
        pltpu.VMEM((BPW, T), jnp.int32),
        pltpu.VMEM((2, T, D), jnp.float32),
        pltpu.VMEM((2, T, D), jnp.float32),
        pltpu.VMEM((T, D), jnp.float32),
        pltpu.VMEM((3, D), jnp.float32),
        pltpu.VMEM((BPW, D), jnp.float32),
        pltpu.SemaphoreType.DMA((2,)),
        pltpu.SemaphoreType.DMA((2,)),
    ],
)
def _embed(smis, char, zeo2, syn2, pe2, te, out, zeo_o, syn_o,
           idx_v, buf_v, obuf_v, pe_v, te_v, zs_v, gsem, osem):
    _body(smis, char, zeo2, syn2, pe2, te, out, zeo_o, syn_o,
          idx_v, buf_v, obuf_v, pe_v, te_v, zs_v, gsem, osem)


def kernel(zeo, syn, smis_seq, char_embed, type_embed, pe):
    b, t = smis_seq.shape
    d = char_embed.shape[1]
    zeo2 = zeo.reshape(b, d)
    syn2 = syn.reshape(b, d)
    pe2 = pe.reshape(t, d)
    out, zeo_o, syn_o = _embed(smis_seq, char_embed, zeo2, syn2, pe2,
                               type_embed)
    return out, zeo_o.reshape(b, 1, d), syn_o.reshape(b, 1, d)


# speedup vs baseline: 1.5821x; 1.1018x over previous
import functools

import jax
import jax.numpy as jnp
from jax import lax
from jax.experimental import pallas as pl
from jax.experimental.pallas import tpu as pltpu
from jax.experimental.pallas import tpu_sc as plsc

B = 4096
T = 125
D = 64
NC = 2
NS = 16
NW = NC * NS
BPW = B // NW
KV = D // 16


def _body(smis, char, zeo2, syn2, pe2, te,
          out, zeo_o, syn_o,
          idx_v, buf_v, obuf_v, pe_v, te_v, zs_v, gsem, osem):
    cid = lax.axis_index("c")
    sid = lax.axis_index("s")
    wid = sid * NC + cid
    base = wid * BPW

    pltpu.sync_copy(smis.at[pl.ds(base, BPW)], idx_v)
    pltpu.sync_copy(pe2, pe_v)
    pltpu.sync_copy(te, te_v)

    def pe_row(pr, c):
        for k in range(KV):
            sl = pl.ds(k * 16, 16)
            pe_v[pr, sl] = pe_v[pr, sl] + te_v[2, sl]
        return c
    lax.fori_loop(0, T, pe_row, 0)

    for src, dst, trow in ((zeo2, zeo_o, 0), (syn2, syn_o, 1)):
        pltpu.sync_copy(src.at[pl.ds(base, BPW)], zs_v)

        def zrow(i, c, trow=trow):
            for k in range(KV):
                sl = pl.ds(k * 16, 16)
                zs_v[i, sl] = zs_v[i, sl] + te_v[trow, sl]
            return c
        lax.fori_loop(0, BPW, zrow, 0)
        pltpu.sync_copy(zs_v, dst.at[pl.ds(base, BPW)])

    def g_start(r, b):
        pltpu.make_async_copy(char.at[idx_v.at[r]], buf_v.at[b],
                              gsem.at[b]).start()

    def g_wait(r, b):
        pltpu.make_async_copy(char.at[idx_v.at[r]], buf_v.at[b],
                              gsem.at[b]).wait()

    def o_start(r, b):
        pltpu.make_async_copy(obuf_v.at[b], out.at[base + r],
                              osem.at[b]).start()

    def o_wait(r, b):
        pltpu.make_async_copy(obuf_v.at[b], out.at[base + r],
                              osem.at[b]).wait()

    def add_rows(b):
        def add_row(pr, cc):
            for k in range(KV):
                sl = pl.ds(k * 16, 16)
                obuf_v[b, pr, sl] = buf_v[b, pr, sl] + pe_v[pr, sl]
            return cc
        lax.fori_loop(0, T, add_row, 0)

    for b in range(2):
        g_start(b, b)
    for b in range(2):
        g_wait(b, b)
        add_rows(b)
        g_start(2 + b, b)
        o_start(b, b)

    def pair(j, c):
        for b in range(2):
            r = 2 * j + b
            o_wait(r - 2, b)
            g_wait(r, b)
            add_rows(b)

            @pl.when(j < BPW // 2 - 1)
            def _():
                g_start(r + 2, b)
            o_start(r, b)
        return c
    lax.fori_loop(1, BPW // 2, pair, 0)
    for b in range(2):
        o_wait(BPW - 2 + b, b)


@functools.partial(
    pl.kernel,
    mesh=plsc.VectorSubcoreMesh(core_axis_name="c", subcore_axis_name="s"),
    compiler_params=pltpu.CompilerParams(use_tc_tiling_on_sc=True),
    out_type=[
        jax.ShapeDtypeStruct((B, T, D), jnp.float32),
        jax.ShapeDtypeStruct((B, D), jnp.float32),
        jax.ShapeDtypeStruct((B, D), jnp.float32),
    ],
    scratch_types=[
        pltpu.VMEM((BPW, T), jnp.int32),
        pltpu.VMEM((2, T, 2 * D), jnp.float32),
        pltpu.VMEM((2, T, D), jnp.float32),
        pltpu.VMEM((T, D), jnp.float32),
        pltpu.VMEM((3, D), jnp.float32),
        pltpu.VMEM((BPW, D), jnp.float32),
        pltpu.SemaphoreType.DMA((2,)),
        pltpu.SemaphoreType.DMA((2,)),
    ],
)
def _embed(smis, char, zeo2, syn2, pe2, te, out, zeo_o, syn_o,
           idx_v, buf_v, obuf_v, pe_v, te_v, zs_v, gsem, osem):
    _body(smis, char, zeo2, syn2, pe2, te, out, zeo_o, syn_o,
          idx_v, buf_v, obuf_v, pe_v, te_v, zs_v, gsem, osem)


def kernel(zeo, syn, smis_seq, char_embed, type_embed, pe):
    b, t = smis_seq.shape
    d = char_embed.shape[1]
    zeo2 = zeo.reshape(b, d)
    syn2 = syn.reshape(b, d)
    pe2 = pe.reshape(t, d)
    char128 = jnp.pad(char_embed, ((0, 0), (0, 128 - d)))
    out, zeo_o, syn_o = _embed(smis_seq, char128, zeo2, syn2, pe2,
                               type_embed)
    return out, zeo_o.reshape(b, 1, d), syn_o.reshape(b, 1, d)
